# baseline (speedup 1.0000x reference)
import functools

import jax
import jax.numpy as jnp
from jax import lax
from jax.experimental import pallas as pl
from jax.experimental.pallas import tpu as pltpu
from jax.experimental.pallas import tpu_sc as plsc

N = 50000
D = 256
B = 512
K = 80
T = N // K
NC, NS, L = 2, 16, 16
NW = NC * NS
NRG = 8
NCG = 4
CD = D // NCG
TPR = (T + NRG - 1) // NRG
RES = T - (NRG - 1) * TPR
TPW = (T + NW - 1) // NW
RESW = T - (NW - 1) * TPW


def _splat(vec, k):
    return lax.gather(
        vec, jnp.full((L, 1), k, jnp.int32),
        lax.GatherDimensionNumbers(offset_dims=(), collapsed_slice_dims=(0,),
                                   start_index_map=(0,)),
        slice_sizes=(1,), mode=lax.GatherScatterMode.PROMISE_IN_BOUNDS)


def _seg_stats_body(x_hbm, idx_hbm, psum_hbm, psq_hbm, pcnt_hbm,
                    idx_v, x0, x1, acc_s, acc_q, acc_c, sem0, sem1):
    cid = lax.axis_index("c")
    sid = lax.axis_index("s")
    wid = cid * NS + sid
    rg = wid // NCG
    cg = wid % NCG
    c0 = cg * CD

    start = rg * TPR
    cnt = jnp.where(rg < NRG - 1, TPR, RES)
    xb = (x0, x1)
    sems = (sem0, sem1)

    def issue(tau, b):
        pltpu.async_copy(x_hbm.at[pl.ds((start + tau) * K, K)], xb[b], sems[b])

    def wait(b):
        pltpu.make_async_copy(x_hbm.at[pl.ds(0, K)], xb[b], sems[b]).wait()

    issue(0, 0)
    pltpu.async_copy(idx_hbm.at[pl.ds(start * K, RES * K)],
                     idx_v.at[pl.ds(0, RES * K)], sem1)
    pltpu.make_async_copy(idx_hbm.at[pl.ds(0, RES * K)],
                          idx_v.at[pl.ds(0, RES * K)], sem1).wait()

    @pl.when(rg < NRG - 1)
    def _():
        rem = (TPR - RES) * K
        pltpu.async_copy(idx_hbm.at[pl.ds(start * K + RES * K, rem)],
                         idx_v.at[pl.ds(RES * K, rem)], sem1)
        pltpu.make_async_copy(idx_hbm.at[pl.ds(0, rem)],
                              idx_v.at[pl.ds(RES * K, rem)], sem1).wait()

    def zrow(i, _):
        def zcol(j, _):
            acc_s[i, pl.ds(j * L, L)] = jnp.zeros((L,), jnp.float32)
            acc_q[i, pl.ds(j * L, L)] = jnp.zeros((L,), jnp.float32)
            return None
        return lax.fori_loop(0, (2 * CD) // L, zcol, None)
    lax.fori_loop(0, B // 2, zrow, None)

    def zc(i, _):
        acc_c[pl.ds(i * L, L)] = jnp.zeros((L,), jnp.float32)
        return None
    lax.fori_loop(0, B // L, zc, None)

    ones16 = jnp.ones((L,), jnp.float32)
    lane0 = lax.iota(jnp.int32, L) == 0

    def _tree(vals):
        while len(vals) > 1:
            vals = [vals[a] + vals[a + 1] for a in range(0, len(vals), 2)]
        return vals[0]

    NJ = CD // L

    def _flush(cur_g, rcnt, svecs, qvecs):
        g2v = jnp.full((L,), lax.shift_right_logical(cur_g, 1), jnp.int32)
        cbase = lax.shift_left(lax.bitwise_and(cur_g, 1), 6)
        for j in range(NJ):
            cols = lax.iota(jnp.int32, L) + j * L + cbase
            plsc.addupdate_scatter(acc_s, [g2v, cols], svecs[j])
            plsc.addupdate_scatter(acc_q, [g2v, cols], qvecs[j])

        @pl.when(cg == 0)
        def _():
            gv = jnp.full((L,), cur_g, jnp.int32)
            plsc.addupdate_scatter(acc_c, [gv], jnp.full((L,), rcnt), mask=lane0)

    def process(tau, b):
        x_v = xb[b]
        zero = jnp.zeros((L,), jnp.float32)

        ivs, uniforms, gfs, TS, TQ = [], [], [], [], []
        for i16 in range(K // L):
            iv = idx_v[pl.ds(tau * K + i16 * L, L)]
            ivs.append(iv)
            g_first = _splat(iv, 0)
            uniforms.append(jnp.all(g_first == _splat(iv, L - 1)))
            gfs.append(jnp.max(g_first))
            ts, tq = [], []
            for j in range(NJ):
                vs = [x_v[i16 * L + r, pl.ds(c0 + j * L, L)] for r in range(L)]
                ts.append(_tree(vs))
                tq.append(_tree([v * v for v in vs]))
            TS.append(ts)
            TQ.append(tq)

        cur_g = jnp.int32(-1)
        rcnt = jnp.float32(0.0)
        svecs = [zero] * NJ
        qvecs = [zero] * NJ
        for i16 in range(K // L):
            iv, uniform, gf = ivs[i16], uniforms[i16], gfs[i16]
            same = jnp.logical_and(uniform, gf == cur_g)

            @pl.when(jnp.logical_and(cur_g >= 0, jnp.logical_not(same)))
            def _(cur_g=cur_g, rcnt=rcnt, svecs=svecs, qvecs=qvecs):
                _flush(cur_g, rcnt, svecs, qvecs)

            @pl.when(jnp.logical_not(uniform))
            def _(iv=iv, i16=i16):
                def srow(k, _):
                    g = _splat(iv, k)
                    r = i16 * L + k
                    g2 = lax.shift_right_logical(g, 1)
                    cbase = lax.shift_left(lax.bitwise_and(g, 1), 6)
                    for j in range(NJ):
                        cols = lax.iota(jnp.int32, L) + j * L + cbase
                        v = x_v[r, pl.ds(c0 + j * L, L)]
                        plsc.addupdate_scatter(acc_s, [g2, cols], v)
                        plsc.addupdate_scatter(acc_q, [g2, cols], v * v)

                    @pl.when(cg == 0)
                    def _():
                        plsc.addupdate_scatter(acc_c, [g], ones16, mask=lane0)
                    return None
                lax.fori_loop(0, L, srow, None)

            ts, tq = TS[i16], TQ[i16]
            rcnt = jnp.where(same, rcnt + 16.0,
                             jnp.where(uniform, 16.0, 0.0))
            svecs = [jnp.where(same, svecs[j] + ts[j],
                               jnp.where(uniform, ts[j], zero))
                     for j in range(NJ)]
            qvecs = [jnp.where(same, qvecs[j] + tq[j],
                               jnp.where(uniform, tq[j], zero))
                     for j in range(NJ)]
            cur_g = jnp.where(uniform, gf, jnp.int32(-1))

        @pl.when(cur_g >= 0)
        def _():
            _flush(cur_g, rcnt, svecs, qvecs)

    def step(tt, _):
        for b in range(2):
            tau = tt * 2 + b

            @pl.when(tau + 1 < cnt)
            def _():
                issue(tau + 1, 1 - b)

            @pl.when(tau < cnt)
            def _():
                wait(b)
                process(tau, b)
        return None
    lax.fori_loop(0, (TPR + 1) // 2, step, None)

    pltpu.sync_copy(acc_s, psum_hbm.at[wid])
    pltpu.sync_copy(acc_q, psq_hbm.at[wid])

    @pl.when(cg == 0)
    def _():
        pltpu.sync_copy(acc_c, pcnt_hbm.at[rg])


def _affine_body(sum_ref, sq_ref, cnt_ref, w_ref, b_ref, scale_ref, shift_ref):
    c = jnp.sum(cnt_ref[...], axis=0)[:, None]
    scales, shifts = [], []
    for cg in range(NCG):
        s = sum_ref[cg]
        q = sq_ref[cg]
        for rg in range(1, NRG):
            s = s + sum_ref[rg * NCG + cg]
            q = q + sq_ref[rg * NCG + cg]
        mean = s / c
        var = jnp.maximum(q / c - mean * mean, 0.0)
        rstd = lax.rsqrt(var + 1e-6)
        sc = rstd * w_ref[0:1, cg * CD:(cg + 1) * CD]
        scales.append(sc)
        shifts.append(b_ref[0:1, cg * CD:(cg + 1) * CD] - mean * sc)
    scale_ref[...] = jnp.concatenate(scales, axis=1)
    shift_ref[...] = jnp.concatenate(shifts, axis=1)


_affine_params = pl.pallas_call(
    _affine_body,
    out_shape=(
        jax.ShapeDtypeStruct((B, D), jnp.float32),
        jax.ShapeDtypeStruct((B, D), jnp.float32),
    ),
)


def _apply_norm_body(x_hbm, idx_hbm, scale_hbm, shift_hbm, out_hbm,
                     idx_v, x0, x1, s0, s1, t0, t1,
                     semi0, semi1, semo0, semo1):
    cid = lax.axis_index("c")
    sid = lax.axis_index("s")
    wid = cid * NS + sid
    base0 = wid * TPW * K
    cnt = jnp.where(wid < NW - 1, TPW, RESW)
    xb, sb, tb = (x0, x1), (s0, s1), (t0, t1)
    semi = (semi0, semi1)
    semo = (semo0, semo1)

    pltpu.async_copy(idx_hbm.at[pl.ds(base0, RESW * K)],
                     idx_v.at[pl.ds(0, RESW * K)], semi1)
    pltpu.make_async_copy(idx_hbm.at[pl.ds(0, RESW * K)],
                          idx_v.at[pl.ds(0, RESW * K)], semi1).wait()

    @pl.when(wid < NW - 1)
    def _():
        rem = (TPW - RESW) * K
        pltpu.async_copy(idx_hbm.at[pl.ds(base0 + RESW * K, rem)],
                         idx_v.at[pl.ds(RESW * K, rem)], semi1)
        pltpu.make_async_copy(idx_hbm.at[pl.ds(0, rem)],
                              idx_v.at[pl.ds(RESW * K, rem)], semi1).wait()

    def issue(tau, b):
        rows = pl.ds(base0 + tau * K, K)
        isl = idx_v.at[pl.ds(tau * K, K)]
        pltpu.async_copy(x_hbm.at[rows], xb[b], semi[b])
        pltpu.async_copy(scale_hbm.at[isl], sb[b], semi[b])
        pltpu.async_copy(shift_hbm.at[isl], tb[b], semi[b])

    def wait_in(b):
        src = x_hbm.at[pl.ds(0, K)]
        pltpu.make_async_copy(src, xb[b], semi[b]).wait()
        pltpu.make_async_copy(src, sb[b], semi[b]).wait()
        pltpu.make_async_copy(src, tb[b], semi[b]).wait()

    def wait_out(b):
        pltpu.make_async_copy(sb[b], out_hbm.at[pl.ds(0, K)], semo[b]).wait()

    issue(0, 0)

    def step(tt, _):
        for b in range(2):
            tau = tt * 2 + b

            @pl.when(tau + 1 < cnt)
            def _():
                @pl.when(tau >= 1)
                def _():
                    wait_out(1 - b)
                issue(tau + 1, 1 - b)

            @pl.when(tau < cnt)
            def _():
                wait_in(b)

                def row(i, _):
                    for j in range(D // L):
                        sl = pl.ds(j * L, L)
                        sb[b][i, sl] = xb[b][i, sl] * sb[b][i, sl] + tb[b][i, sl]
                    return None
                lax.fori_loop(0, K, row, None)
                pltpu.async_copy(sb[b], out_hbm.at[pl.ds(base0 + tau * K, K)],
                                 semo[b])
        return None
    lax.fori_loop(0, (TPW + 1) // 2, step, None)

    wait_out(0)
    wait_out(1)


@jax.jit
def kernel(tensor, weight, bias, batch_index):
    mesh = plsc.VectorSubcoreMesh(
        core_axis_name="c", subcore_axis_name="s", num_cores=NC, num_subcores=NS
    )
    seg_stats = functools.partial(
        pl.kernel,
        out_type=(
            jax.ShapeDtypeStruct((NW, B // 2, 2 * CD), jnp.float32),
            jax.ShapeDtypeStruct((NW, B // 2, 2 * CD), jnp.float32),
            jax.ShapeDtypeStruct((NRG, B), jnp.float32),
        ),
        mesh=mesh,
        compiler_params=pltpu.CompilerParams(needs_layout_passes=False),
        scratch_types=[
            pltpu.VMEM((TPR * K,), jnp.int32),
            pltpu.VMEM((K, D), jnp.float32),
            pltpu.VMEM((K, D), jnp.float32),
            pltpu.VMEM((B // 2, 2 * CD), jnp.float32),
            pltpu.VMEM((B // 2, 2 * CD), jnp.float32),
            pltpu.VMEM((B,), jnp.float32),
            pltpu.SemaphoreType.DMA,
            pltpu.SemaphoreType.DMA,
        ],
    )(_seg_stats_body)
    apply_norm = functools.partial(
        pl.kernel,
        out_type=jax.ShapeDtypeStruct((N, D), jnp.float32),
        mesh=mesh,
        scratch_types=[
            pltpu.VMEM((TPW * K,), jnp.int32),
            pltpu.VMEM((K, D), jnp.float32),
            pltpu.VMEM((K, D), jnp.float32),
            pltpu.VMEM((K, D), jnp.float32),
            pltpu.VMEM((K, D), jnp.float32),
            pltpu.VMEM((K, D), jnp.float32),
            pltpu.VMEM((K, D), jnp.float32),
            pltpu.SemaphoreType.DMA,
            pltpu.SemaphoreType.DMA,
            pltpu.SemaphoreType.DMA,
            pltpu.SemaphoreType.DMA,
        ],
    )(_apply_norm_body)

    idx = batch_index.astype(jnp.int32)
    psum_raw, psq_raw, pcnt = seg_stats(tensor, idx)
    psum = psum_raw.reshape(NW, B, CD)
    psq = psq_raw.reshape(NW, B, CD)
    scale, shift = _affine_params(psum, psq, pcnt,
                                  weight.reshape(1, D), bias.reshape(1, D))
    return apply_norm(tensor, idx, scale, shift)


# speedup vs baseline: 1.2926x; 1.2926x over previous
import functools

import jax
import jax.numpy as jnp
from jax import lax
from jax.experimental import pallas as pl
from jax.experimental.pallas import tpu as pltpu
from jax.experimental.pallas import tpu_sc as plsc

N = 50000
D = 256
B = 512
K = 80
T = N // K
NC, NS, L = 2, 16, 16
NW = NC * NS
NRG = 8
NCG = 4
CD = D // NCG
TPR = (T + NRG - 1) // NRG
RES = T - (NRG - 1) * TPR
TPW = (T + NW - 1) // NW
RESW = T - (NW - 1) * TPW


def _splat(vec, k):
    return lax.gather(
        vec, jnp.full((L, 1), k, jnp.int32),
        lax.GatherDimensionNumbers(offset_dims=(), collapsed_slice_dims=(0,),
                                   start_index_map=(0,)),
        slice_sizes=(1,), mode=lax.GatherScatterMode.PROMISE_IN_BOUNDS)


def _seg_stats_body(x_hbm, idx_hbm, psum_hbm, psq_hbm, pcnt_hbm,
                    idx_v, x0, x1, acc_s, acc_q, acc_c, sem0, sem1):
    cid = lax.axis_index("c")
    sid = lax.axis_index("s")
    wid = cid * NS + sid
    rg = wid // NCG
    cg = wid % NCG
    c0 = cg * CD

    start = rg * TPR
    cnt = jnp.where(rg < NRG - 1, TPR, RES)
    xb = (x0, x1)
    sems = (sem0, sem1)

    def issue(tau, b):
        pltpu.async_copy(x_hbm.at[pl.ds((start + tau) * K, K)], xb[b], sems[b])

    def wait(b):
        pltpu.make_async_copy(x_hbm.at[pl.ds(0, K)], xb[b], sems[b]).wait()

    issue(0, 0)
    pltpu.async_copy(idx_hbm.at[pl.ds(start * K, RES * K)],
                     idx_v.at[pl.ds(0, RES * K)], sem1)
    pltpu.make_async_copy(idx_hbm.at[pl.ds(0, RES * K)],
                          idx_v.at[pl.ds(0, RES * K)], sem1).wait()

    @pl.when(rg < NRG - 1)
    def _():
        rem = (TPR - RES) * K
        pltpu.async_copy(idx_hbm.at[pl.ds(start * K + RES * K, rem)],
                         idx_v.at[pl.ds(RES * K, rem)], sem1)
        pltpu.make_async_copy(idx_hbm.at[pl.ds(0, rem)],
                              idx_v.at[pl.ds(RES * K, rem)], sem1).wait()

    def zrow(i, _):
        def zcol(j, _):
            acc_s[i, pl.ds(j * L, L)] = jnp.zeros((L,), jnp.float32)
            acc_q[i, pl.ds(j * L, L)] = jnp.zeros((L,), jnp.float32)
            return None
        return lax.fori_loop(0, (2 * CD) // L, zcol, None)
    lax.fori_loop(0, B // 2, zrow, None)

    def zc(i, _):
        acc_c[pl.ds(i * L, L)] = jnp.zeros((L,), jnp.float32)
        return None
    lax.fori_loop(0, B // L, zc, None)

    ones16 = jnp.ones((L,), jnp.float32)
    lane0 = lax.iota(jnp.int32, L) == 0

    def _tree(vals):
        while len(vals) > 1:
            vals = [vals[a] + vals[a + 1] for a in range(0, len(vals), 2)]
        return vals[0]

    NJ = CD // L

    def _flush(cur_g, rcnt, svecs, qvecs):
        g2v = jnp.full((L,), lax.shift_right_logical(cur_g, 1), jnp.int32)
        cbase = lax.shift_left(lax.bitwise_and(cur_g, 1), 6)
        for j in range(NJ):
            cols = lax.iota(jnp.int32, L) + j * L + cbase
            plsc.addupdate_scatter(acc_s, [g2v, cols], svecs[j])
            plsc.addupdate_scatter(acc_q, [g2v, cols], qvecs[j])

        @pl.when(cg == 0)
        def _():
            gv = jnp.full((L,), cur_g, jnp.int32)
            plsc.addupdate_scatter(acc_c, [gv], jnp.full((L,), rcnt), mask=lane0)

    def process(tau, b):
        x_v = xb[b]
        zero = jnp.zeros((L,), jnp.float32)
        init = (jnp.int32(-1), jnp.float32(0.0)) + (zero,) * (2 * NJ)

        def grp(i16, carry):
            cur_g, rcnt = carry[0], carry[1]
            svecs, qvecs = carry[2:2 + NJ], carry[2 + NJ:]
            iv = idx_v[pl.ds(tau * K + i16 * L, L)]
            g_first = _splat(iv, 0)
            uniform = jnp.all(g_first == _splat(iv, L - 1))
            gf = jnp.max(g_first)
            same = jnp.logical_and(uniform, gf == cur_g)

            ts, tq = [], []
            for j in range(NJ):
                vs = [x_v[i16 * L + r, pl.ds(c0 + j * L, L)] for r in range(L)]
                ts.append(_tree(vs))
                tq.append(_tree([v * v for v in vs]))

            @pl.when(jnp.logical_and(cur_g >= 0, jnp.logical_not(same)))
            def _():
                _flush(cur_g, rcnt, svecs, qvecs)

            @pl.when(jnp.logical_not(uniform))
            def _():
                for k in range(L):
                    g = _splat(iv, k)
                    r = i16 * L + k
                    g2 = lax.shift_right_logical(g, 1)
                    cbase = lax.shift_left(lax.bitwise_and(g, 1), 6)
                    for j in range(NJ):
                        cols = lax.iota(jnp.int32, L) + j * L + cbase
                        v = x_v[r, pl.ds(c0 + j * L, L)]
                        plsc.addupdate_scatter(acc_s, [g2, cols], v)
                        plsc.addupdate_scatter(acc_q, [g2, cols], v * v)

                    @pl.when(cg == 0)
                    def _():
                        plsc.addupdate_scatter(acc_c, [g], ones16, mask=lane0)

            new_g = jnp.where(uniform, gf, jnp.int32(-1))
            new_cnt = jnp.where(same, rcnt + 16.0,
                                jnp.where(uniform, 16.0, 0.0))
            new_s = [jnp.where(same, svecs[j] + ts[j],
                               jnp.where(uniform, ts[j], zero))
                     for j in range(NJ)]
            new_q = [jnp.where(same, qvecs[j] + tq[j],
                               jnp.where(uniform, tq[j], zero))
                     for j in range(NJ)]
            return (new_g, new_cnt) + tuple(new_s) + tuple(new_q)

        def grp_pl(i16, carry):
            return grp(i16, carry)
        fin = plsc.parallel_loop(0, K // L, carry=init)(grp_pl)

        @pl.when(fin[0] >= 0)
        def _():
            _flush(fin[0], fin[1], fin[2:2 + NJ], fin[2 + NJ:])

    def step(tt, _):
        for b in range(2):
            tau = tt * 2 + b

            @pl.when(tau + 1 < cnt)
            def _():
                issue(tau + 1, 1 - b)

            @pl.when(tau < cnt)
            def _():
                wait(b)
                process(tau, b)
        return None
    lax.fori_loop(0, (TPR + 1) // 2, step, None)

    pltpu.sync_copy(acc_s, psum_hbm.at[wid])
    pltpu.sync_copy(acc_q, psq_hbm.at[wid])

    @pl.when(cg == 0)
    def _():
        pltpu.sync_copy(acc_c, pcnt_hbm.at[rg])


def _affine_body(sum_ref, sq_ref, cnt_ref, w_ref, b_ref, scale_ref, shift_ref):
    c = jnp.sum(cnt_ref[...], axis=0)[:, None]
    scales, shifts = [], []
    for cg in range(NCG):
        s = sum_ref[cg]
        q = sq_ref[cg]
        for rg in range(1, NRG):
            s = s + sum_ref[rg * NCG + cg]
            q = q + sq_ref[rg * NCG + cg]
        mean = s / c
        var = jnp.maximum(q / c - mean * mean, 0.0)
        rstd = lax.rsqrt(var + 1e-6)
        sc = rstd * w_ref[0:1, cg * CD:(cg + 1) * CD]
        scales.append(sc)
        shifts.append(b_ref[0:1, cg * CD:(cg + 1) * CD] - mean * sc)
    scale_ref[...] = jnp.concatenate(scales, axis=1)
    shift_ref[...] = jnp.concatenate(shifts, axis=1)


_affine_params = pl.pallas_call(
    _affine_body,
    out_shape=(
        jax.ShapeDtypeStruct((B, D), jnp.float32),
        jax.ShapeDtypeStruct((B, D), jnp.float32),
    ),
)


def _apply_norm_body(x_hbm, idx_hbm, scale_hbm, shift_hbm, out_hbm,
                     idx_v, x0, x1, s0, s1, t0, t1,
                     semi0, semi1, semo0, semo1):
    cid = lax.axis_index("c")
    sid = lax.axis_index("s")
    wid = cid * NS + sid
    base0 = wid * TPW * K
    cnt = jnp.where(wid < NW - 1, TPW, RESW)
    xb, sb, tb = (x0, x1), (s0, s1), (t0, t1)
    semi = (semi0, semi1)
    semo = (semo0, semo1)

    pltpu.async_copy(idx_hbm.at[pl.ds(base0, RESW * K)],
                     idx_v.at[pl.ds(0, RESW * K)], semi1)
    pltpu.make_async_copy(idx_hbm.at[pl.ds(0, RESW * K)],
                          idx_v.at[pl.ds(0, RESW * K)], semi1).wait()

    @pl.when(wid < NW - 1)
    def _():
        rem = (TPW - RESW) * K
        pltpu.async_copy(idx_hbm.at[pl.ds(base0 + RESW * K, rem)],
                         idx_v.at[pl.ds(RESW * K, rem)], semi1)
        pltpu.make_async_copy(idx_hbm.at[pl.ds(0, rem)],
                              idx_v.at[pl.ds(RESW * K, rem)], semi1).wait()

    def issue(tau, b):
        rows = pl.ds(base0 + tau * K, K)
        isl = idx_v.at[pl.ds(tau * K, K)]
        pltpu.async_copy(x_hbm.at[rows], xb[b], semi[b])
        pltpu.async_copy(scale_hbm.at[isl], sb[b], semi[b])
        pltpu.async_copy(shift_hbm.at[isl], tb[b], semi[b])

    def wait_in(b):
        src = x_hbm.at[pl.ds(0, K)]
        pltpu.make_async_copy(src, xb[b], semi[b]).wait()
        pltpu.make_async_copy(src, sb[b], semi[b]).wait()
        pltpu.make_async_copy(src, tb[b], semi[b]).wait()

    def wait_out(b):
        pltpu.make_async_copy(sb[b], out_hbm.at[pl.ds(0, K)], semo[b]).wait()

    issue(0, 0)

    def step(tt, _):
        for b in range(2):
            tau = tt * 2 + b

            @pl.when(tau + 1 < cnt)
            def _():
                @pl.when(tau >= 1)
                def _():
                    wait_out(1 - b)
                issue(tau + 1, 1 - b)

            @pl.when(tau < cnt)
            def _():
                wait_in(b)

                @plsc.parallel_loop(0, K, unroll=4)
                def _(i):
                    for j in range(D // L):
                        sl = pl.ds(j * L, L)
                        sb[b][i, sl] = xb[b][i, sl] * sb[b][i, sl] + tb[b][i, sl]
                pltpu.async_copy(sb[b], out_hbm.at[pl.ds(base0 + tau * K, K)],
                                 semo[b])
        return None
    lax.fori_loop(0, (TPW + 1) // 2, step, None)

    wait_out(0)
    wait_out(1)


@jax.jit
def kernel(tensor, weight, bias, batch_index):
    mesh = plsc.VectorSubcoreMesh(
        core_axis_name="c", subcore_axis_name="s", num_cores=NC, num_subcores=NS
    )
    seg_stats = functools.partial(
        pl.kernel,
        out_type=(
            jax.ShapeDtypeStruct((NW, B // 2, 2 * CD), jnp.float32),
            jax.ShapeDtypeStruct((NW, B // 2, 2 * CD), jnp.float32),
            jax.ShapeDtypeStruct((NRG, B), jnp.float32),
        ),
        mesh=mesh,
        compiler_params=pltpu.CompilerParams(needs_layout_passes=False),
        scratch_types=[
            pltpu.VMEM((TPR * K,), jnp.int32),
            pltpu.VMEM((K, D), jnp.float32),
            pltpu.VMEM((K, D), jnp.float32),
            pltpu.VMEM((B // 2, 2 * CD), jnp.float32),
            pltpu.VMEM((B // 2, 2 * CD), jnp.float32),
            pltpu.VMEM((B,), jnp.float32),
            pltpu.SemaphoreType.DMA,
            pltpu.SemaphoreType.DMA,
        ],
    )(_seg_stats_body)
    apply_norm = functools.partial(
        pl.kernel,
        out_type=jax.ShapeDtypeStruct((N, D), jnp.float32),
        mesh=mesh,
        scratch_types=[
            pltpu.VMEM((TPW * K,), jnp.int32),
            pltpu.VMEM((K, D), jnp.float32),
            pltpu.VMEM((K, D), jnp.float32),
            pltpu.VMEM((K, D), jnp.float32),
            pltpu.VMEM((K, D), jnp.float32),
            pltpu.VMEM((K, D), jnp.float32),
            pltpu.VMEM((K, D), jnp.float32),
            pltpu.SemaphoreType.DMA,
            pltpu.SemaphoreType.DMA,
            pltpu.SemaphoreType.DMA,
            pltpu.SemaphoreType.DMA,
        ],
    )(_apply_norm_body)

    idx = batch_index.astype(jnp.int32)
    psum_raw, psq_raw, pcnt = seg_stats(tensor, idx)
    psum = psum_raw.reshape(NW, B, CD)
    psq = psq_raw.reshape(NW, B, CD)
    scale, shift = _affine_params(psum, psq, pcnt,
                                  weight.reshape(1, D), bias.reshape(1, D))
    return apply_norm(tensor, idx, scale, shift)


# speedup vs baseline: 2.1762x; 1.6835x over previous
import functools

import jax
import jax.numpy as jnp
from jax import lax
from jax.experimental import pallas as pl
from jax.experimental.pallas import tpu as pltpu
from jax.experimental.pallas import tpu_sc as plsc

N = 50000
D = 256
B = 512
K = 80
T = N // K
NC, NS, L = 2, 16, 16
NW = NC * NS
NRG = 8
NCG = 4
CD = D // NCG
TPR = (T + NRG - 1) // NRG
RES = T - (NRG - 1) * TPR
TPW = (T + NW - 1) // NW
RESW = T - (NW - 1) * TPW


def _splat(vec, k):
    return lax.gather(
        vec, jnp.full((L, 1), k, jnp.int32),
        lax.GatherDimensionNumbers(offset_dims=(), collapsed_slice_dims=(0,),
                                   start_index_map=(0,)),
        slice_sizes=(1,), mode=lax.GatherScatterMode.PROMISE_IN_BOUNDS)


def _seg_stats_body(x_hbm, idx_hbm, psum_hbm, psq_hbm, pcnt_hbm,
                    idx_v, x0, x1, acc_s, acc_q, acc_c, sem0, sem1):
    cid = lax.axis_index("c")
    sid = lax.axis_index("s")
    wid = cid * NS + sid
    rg = wid // NCG
    cg = wid % NCG
    c0 = cg * CD

    start = rg * TPR
    cnt = jnp.where(rg < NRG - 1, TPR, RES)
    xb = (x0, x1)
    sems = (sem0, sem1)

    def issue(tau, b):
        pltpu.async_copy(x_hbm.at[pl.ds((start + tau) * K, K)], xb[b], sems[b])

    def wait(b):
        pltpu.make_async_copy(x_hbm.at[pl.ds(0, K)], xb[b], sems[b]).wait()

    issue(0, 0)
    pltpu.async_copy(idx_hbm.at[pl.ds(start * K, RES * K)],
                     idx_v.at[pl.ds(0, RES * K)], sem1)
    pltpu.make_async_copy(idx_hbm.at[pl.ds(0, RES * K)],
                          idx_v.at[pl.ds(0, RES * K)], sem1).wait()

    @pl.when(rg < NRG - 1)
    def _():
        rem = (TPR - RES) * K
        pltpu.async_copy(idx_hbm.at[pl.ds(start * K + RES * K, rem)],
                         idx_v.at[pl.ds(RES * K, rem)], sem1)
        pltpu.make_async_copy(idx_hbm.at[pl.ds(0, rem)],
                              idx_v.at[pl.ds(RES * K, rem)], sem1).wait()

    def zrow(i, _):
        def zcol(j, _):
            acc_s[i, pl.ds(j * L, L)] = jnp.zeros((L,), jnp.float32)
            acc_q[i, pl.ds(j * L, L)] = jnp.zeros((L,), jnp.float32)
            return None
        return lax.fori_loop(0, (2 * CD) // L, zcol, None)
    lax.fori_loop(0, B // 2, zrow, None)

    def zc(i, _):
        acc_c[pl.ds(i * L, L)] = jnp.zeros((L,), jnp.float32)
        return None
    lax.fori_loop(0, B // L, zc, None)

    ones16 = jnp.ones((L,), jnp.float32)
    lane0 = lax.iota(jnp.int32, L) == 0

    def _tree(vals):
        while len(vals) > 1:
            vals = [vals[a] + vals[a + 1] for a in range(0, len(vals), 2)]
        return vals[0]

    NJ = CD // L

    def _flush(cur_g, rcnt, svecs, qvecs):
        g2v = jnp.full((L,), lax.shift_right_logical(cur_g, 1), jnp.int32)
        cbase = lax.shift_left(lax.bitwise_and(cur_g, 1), 6)
        for j in range(NJ):
            cols = lax.iota(jnp.int32, L) + j * L + cbase
            plsc.addupdate_scatter(acc_s, [g2v, cols], svecs[j])
            plsc.addupdate_scatter(acc_q, [g2v, cols], qvecs[j])

        @pl.when(cg == 0)
        def _():
            gv = jnp.full((L,), cur_g, jnp.int32)
            plsc.addupdate_scatter(acc_c, [gv], jnp.full((L,), rcnt), mask=lane0)

    def process(tau, b):
        x_v = xb[b]
        zero = jnp.zeros((L,), jnp.float32)
        init = (jnp.int32(-1), jnp.float32(0.0)) + (zero,) * (2 * NJ)

        def grp(i16, carry):
            cur_g, rcnt = carry[0], carry[1]
            svecs, qvecs = carry[2:2 + NJ], carry[2 + NJ:]
            iv = idx_v[pl.ds(tau * K + i16 * L, L)]
            g_first = _splat(iv, 0)
            uniform = jnp.all(g_first == _splat(iv, L - 1))
            gf = jnp.max(g_first)
            same = jnp.logical_and(uniform, gf == cur_g)

            ts, tq = [], []
            for j in range(NJ):
                vs = [x_v[i16 * L + r, pl.ds(c0 + j * L, L)] for r in range(L)]
                ts.append(_tree(vs))
                tq.append(_tree([v * v for v in vs]))

            @pl.when(jnp.logical_and(cur_g >= 0, jnp.logical_not(same)))
            def _():
                _flush(cur_g, rcnt, svecs, qvecs)

            @pl.when(jnp.logical_not(uniform))
            def _():
                for k in range(L):
                    g = _splat(iv, k)
                    r = i16 * L + k
                    g2 = lax.shift_right_logical(g, 1)
                    cbase = lax.shift_left(lax.bitwise_and(g, 1), 6)
                    for j in range(NJ):
                        cols = lax.iota(jnp.int32, L) + j * L + cbase
                        v = x_v[r, pl.ds(c0 + j * L, L)]
                        plsc.addupdate_scatter(acc_s, [g2, cols], v)
                        plsc.addupdate_scatter(acc_q, [g2, cols], v * v)

                    @pl.when(cg == 0)
                    def _():
                        plsc.addupdate_scatter(acc_c, [g], ones16, mask=lane0)

            new_g = jnp.where(uniform, gf, jnp.int32(-1))
            new_cnt = jnp.where(same, rcnt + 16.0,
                                jnp.where(uniform, 16.0, 0.0))
            new_s = [jnp.where(same, svecs[j] + ts[j],
                               jnp.where(uniform, ts[j], zero))
                     for j in range(NJ)]
            new_q = [jnp.where(same, qvecs[j] + tq[j],
                               jnp.where(uniform, tq[j], zero))
                     for j in range(NJ)]
            return (new_g, new_cnt) + tuple(new_s) + tuple(new_q)

        def grp_pl(i16, carry):
            return grp(i16, carry)
        fin = plsc.parallel_loop(0, K // L, carry=init)(grp_pl)

        @pl.when(fin[0] >= 0)
        def _():
            _flush(fin[0], fin[1], fin[2:2 + NJ], fin[2 + NJ:])

    def step(tt, _):
        for b in range(2):
            tau = tt * 2 + b

            @pl.when(tau + 1 < cnt)
            def _():
                issue(tau + 1, 1 - b)

            @pl.when(tau < cnt)
            def _():
                wait(b)
                process(tau, b)
        return None
    lax.fori_loop(0, (TPR + 1) // 2, step, None)

    pltpu.sync_copy(acc_s, psum_hbm.at[wid])
    pltpu.sync_copy(acc_q, psq_hbm.at[wid])

    @pl.when(cg == 0)
    def _():
        pltpu.sync_copy(acc_c, pcnt_hbm.at[rg])


def _affine_body(sum_ref, sq_ref, cnt_ref, w_ref, b_ref, scale_ref, shift_ref):
    c = jnp.sum(cnt_ref[...], axis=0)[:, None]
    scales, shifts = [], []
    for cg in range(NCG):
        s = sum_ref[cg]
        q = sq_ref[cg]
        for rg in range(1, NRG):
            s = s + sum_ref[rg * NCG + cg]
            q = q + sq_ref[rg * NCG + cg]
        mean = s / c
        var = jnp.maximum(q / c - mean * mean, 0.0)
        rstd = lax.rsqrt(var + 1e-6)
        sc = rstd * w_ref[0:1, cg * CD:(cg + 1) * CD]
        scales.append(sc)
        shifts.append(b_ref[0:1, cg * CD:(cg + 1) * CD] - mean * sc)
    scale_ref[...] = jnp.concatenate(scales, axis=1)
    shift_ref[...] = jnp.concatenate(shifts, axis=1)


_affine_params = pl.pallas_call(
    _affine_body,
    out_shape=(
        jax.ShapeDtypeStruct((B, D), jnp.float32),
        jax.ShapeDtypeStruct((B, D), jnp.float32),
    ),
)


def _apply_norm_body(x_hbm, idx_hbm, scale_hbm, shift_hbm, out_hbm,
                     idx_v, x0, x1, s0, s1, t0, t1,
                     semi0, semi1, semo0, semo1):
    cid = lax.axis_index("c")
    sid = lax.axis_index("s")
    wid = cid * NS + sid
    base0 = wid * TPW * K
    cnt = jnp.where(wid < NW - 1, TPW, RESW)
    xb, sb, tb = (x0, x1), (s0, s1), (t0, t1)
    semi = (semi0, semi1)
    semo = (semo0, semo1)

    pltpu.async_copy(idx_hbm.at[pl.ds(base0, RESW * K)],
                     idx_v.at[pl.ds(0, RESW * K)], semi1)
    pltpu.make_async_copy(idx_hbm.at[pl.ds(0, RESW * K)],
                          idx_v.at[pl.ds(0, RESW * K)], semi1).wait()

    @pl.when(wid < NW - 1)
    def _():
        rem = (TPW - RESW) * K
        pltpu.async_copy(idx_hbm.at[pl.ds(base0 + RESW * K, rem)],
                         idx_v.at[pl.ds(RESW * K, rem)], semi1)
        pltpu.make_async_copy(idx_hbm.at[pl.ds(0, rem)],
                              idx_v.at[pl.ds(RESW * K, rem)], semi1).wait()

    W = 16

    def tile_window(tau):
        iv0 = idx_v[pl.ds(tau * K, L)]
        ivl = idx_v[pl.ds(tau * K + K - L, L)]
        first = jnp.max(_splat(iv0, 0))
        last = jnp.max(_splat(ivl, L - 1))
        firstc = jnp.minimum(jnp.bitwise_and(first, -8), B - W)
        return firstc, last - firstc < W

    def issue(tau, b):
        rows = pl.ds(base0 + tau * K, K)
        pltpu.async_copy(x_hbm.at[rows], xb[b], semi[b])
        firstc, ok = tile_window(tau)
        fc = pl.multiple_of(firstc, 8)

        @pl.when(ok)
        def _():
            pltpu.async_copy(scale_hbm.at[pl.ds(fc, W)],
                             sb[b].at[pl.ds(0, W)], semi[b])
            pltpu.async_copy(shift_hbm.at[pl.ds(fc, W)],
                             tb[b].at[pl.ds(0, W)], semi[b])

        @pl.when(jnp.logical_not(ok))
        def _():
            isl = idx_v.at[pl.ds(tau * K, K)]
            pltpu.async_copy(scale_hbm.at[isl], sb[b], semi[b])
            pltpu.async_copy(shift_hbm.at[isl], tb[b], semi[b])

    def wait_in(tau, b):
        src = x_hbm.at[pl.ds(0, K)]
        pltpu.make_async_copy(src, xb[b], semi[b]).wait()
        _, ok = tile_window(tau)

        @pl.when(ok)
        def _():
            srw = x_hbm.at[pl.ds(0, W)]
            pltpu.make_async_copy(srw, sb[b].at[pl.ds(0, W)], semi[b]).wait()
            pltpu.make_async_copy(srw, tb[b].at[pl.ds(0, W)], semi[b]).wait()

        @pl.when(jnp.logical_not(ok))
        def _():
            pltpu.make_async_copy(src, sb[b], semi[b]).wait()
            pltpu.make_async_copy(src, tb[b], semi[b]).wait()

    def wait_out(b):
        pltpu.make_async_copy(xb[b], out_hbm.at[pl.ds(0, K)], semo[b]).wait()

    issue(0, 0)

    def step(tt, _):
        for b in range(2):
            tau = tt * 2 + b

            @pl.when(tau + 1 < cnt)
            def _():
                @pl.when(tau >= 1)
                def _():
                    wait_out(1 - b)
                issue(tau + 1, 1 - b)

            @pl.when(tau < cnt)
            def _():
                wait_in(tau, b)
                firstc, ok = tile_window(tau)

                @plsc.parallel_loop(0, K, unroll=4)
                def _(i):
                    iv16 = idx_v[pl.ds(tau * K + jnp.bitwise_and(i, -16), L)]
                    gs = jnp.max(_splat(iv16, jnp.bitwise_and(i, 15)))
                    rsel = jnp.where(ok, gs - firstc, i)
                    for j in range(D // L):
                        sl = pl.ds(j * L, L)
                        xb[b][i, sl] = (xb[b][i, sl] * sb[b][rsel, sl]
                                        + tb[b][rsel, sl])
                pltpu.async_copy(xb[b], out_hbm.at[pl.ds(base0 + tau * K, K)],
                                 semo[b])
        return None
    lax.fori_loop(0, (TPW + 1) // 2, step, None)

    wait_out(0)
    wait_out(1)


@jax.jit
def kernel(tensor, weight, bias, batch_index):
    mesh = plsc.VectorSubcoreMesh(
        core_axis_name="c", subcore_axis_name="s", num_cores=NC, num_subcores=NS
    )
    seg_stats = functools.partial(
        pl.kernel,
        out_type=(
            jax.ShapeDtypeStruct((NW, B // 2, 2 * CD), jnp.float32),
            jax.ShapeDtypeStruct((NW, B // 2, 2 * CD), jnp.float32),
            jax.ShapeDtypeStruct((NRG, B), jnp.float32),
        ),
        mesh=mesh,
        compiler_params=pltpu.CompilerParams(needs_layout_passes=False),
        scratch_types=[
            pltpu.VMEM((TPR * K,), jnp.int32),
            pltpu.VMEM((K, D), jnp.float32),
            pltpu.VMEM((K, D), jnp.float32),
            pltpu.VMEM((B // 2, 2 * CD), jnp.float32),
            pltpu.VMEM((B // 2, 2 * CD), jnp.float32),
            pltpu.VMEM((B,), jnp.float32),
            pltpu.SemaphoreType.DMA,
            pltpu.SemaphoreType.DMA,
        ],
    )(_seg_stats_body)
    apply_norm = functools.partial(
        pl.kernel,
        out_type=jax.ShapeDtypeStruct((N, D), jnp.float32),
        mesh=mesh,
        compiler_params=pltpu.CompilerParams(needs_layout_passes=False),
        scratch_types=[
            pltpu.VMEM((TPW * K,), jnp.int32),
            pltpu.VMEM((K, D), jnp.float32),
            pltpu.VMEM((K, D), jnp.float32),
            pltpu.VMEM((K, D), jnp.float32),
            pltpu.VMEM((K, D), jnp.float32),
            pltpu.VMEM((K, D), jnp.float32),
            pltpu.VMEM((K, D), jnp.float32),
            pltpu.SemaphoreType.DMA,
            pltpu.SemaphoreType.DMA,
            pltpu.SemaphoreType.DMA,
            pltpu.SemaphoreType.DMA,
        ],
    )(_apply_norm_body)

    idx = batch_index.astype(jnp.int32)
    psum_raw, psq_raw, pcnt = seg_stats(tensor, idx)
    psum = psum_raw.reshape(NW, B, CD)
    psq = psq_raw.reshape(NW, B, CD)
    scale, shift = _affine_params(psum, psq, pcnt,
                                  weight.reshape(1, D), bias.reshape(1, D))
    return apply_norm(tensor, idx, scale, shift)


# speedup vs baseline: 2.4676x; 1.1339x over previous
import functools

import jax
import jax.numpy as jnp
from jax import lax
from jax.experimental import pallas as pl
from jax.experimental.pallas import tpu as pltpu
from jax.experimental.pallas import tpu_sc as plsc

N = 50000
D = 256
B = 512
K = 80
T = N // K
NC, NS, L = 2, 16, 16
NW = NC * NS
NRG = 8
NCG = 4
CD = D // NCG
TPR = (T + NRG - 1) // NRG
RES = T - (NRG - 1) * TPR
TPW = (T + NW - 1) // NW
RESW = T - (NW - 1) * TPW


def _splat(vec, k):
    return lax.gather(
        vec, jnp.full((L, 1), k, jnp.int32),
        lax.GatherDimensionNumbers(offset_dims=(), collapsed_slice_dims=(0,),
                                   start_index_map=(0,)),
        slice_sizes=(1,), mode=lax.GatherScatterMode.PROMISE_IN_BOUNDS)


def _seg_stats_body(x_hbm, idx_hbm, psum_hbm, psq_hbm, pcnt_hbm,
                    idx_v, x0, x1, acc_s, acc_q, acc_c, sem0, sem1):
    cid = lax.axis_index("c")
    sid = lax.axis_index("s")
    wid = cid * NS + sid
    rg = wid // NCG
    cg = wid % NCG
    c0 = lax.bitwise_and(cg, 1) * CD

    start = rg * TPR
    cnt = jnp.where(rg < NRG - 1, TPR, RES)
    xb = (x0, x1)
    sems = (sem0, sem1)

    chalf = pl.multiple_of(lax.shift_right_logical(cg, 1) * (D // 2), D // 2)

    def issue(tau, b):
        pltpu.async_copy(
            x_hbm.at[pl.ds((start + tau) * K, K), pl.ds(chalf, D // 2)],
            xb[b], sems[b])

    def wait(b):
        pltpu.make_async_copy(
            x_hbm.at[pl.ds(0, K), pl.ds(0, D // 2)], xb[b], sems[b]).wait()

    issue(0, 0)
    pltpu.async_copy(idx_hbm.at[pl.ds(start * K, RES * K)],
                     idx_v.at[pl.ds(0, RES * K)], sem1)
    pltpu.make_async_copy(idx_hbm.at[pl.ds(0, RES * K)],
                          idx_v.at[pl.ds(0, RES * K)], sem1).wait()

    @pl.when(rg < NRG - 1)
    def _():
        rem = (TPR - RES) * K
        pltpu.async_copy(idx_hbm.at[pl.ds(start * K + RES * K, rem)],
                         idx_v.at[pl.ds(RES * K, rem)], sem1)
        pltpu.make_async_copy(idx_hbm.at[pl.ds(0, rem)],
                              idx_v.at[pl.ds(RES * K, rem)], sem1).wait()

    def zrow(i, _):
        def zcol(j, _):
            acc_s[i, pl.ds(j * L, L)] = jnp.zeros((L,), jnp.float32)
            acc_q[i, pl.ds(j * L, L)] = jnp.zeros((L,), jnp.float32)
            return None
        return lax.fori_loop(0, (2 * CD) // L, zcol, None)
    lax.fori_loop(0, B // 2, zrow, None)

    def zc(i, _):
        acc_c[pl.ds(i * L, L)] = jnp.zeros((L,), jnp.float32)
        return None
    lax.fori_loop(0, B // L, zc, None)

    ones16 = jnp.ones((L,), jnp.float32)
    lane0 = lax.iota(jnp.int32, L) == 0

    def _tree(vals):
        while len(vals) > 1:
            vals = [vals[a] + vals[a + 1] for a in range(0, len(vals), 2)]
        return vals[0]

    NJ = CD // L

    def _flush(cur_g, rcnt, svecs, qvecs):
        g2v = jnp.full((L,), lax.shift_right_logical(cur_g, 1), jnp.int32)
        cbase = lax.shift_left(lax.bitwise_and(cur_g, 1), 6)
        for j in range(NJ):
            cols = lax.iota(jnp.int32, L) + j * L + cbase
            plsc.addupdate_scatter(acc_s, [g2v, cols], svecs[j])
            plsc.addupdate_scatter(acc_q, [g2v, cols], qvecs[j])

        @pl.when(cg == 0)
        def _():
            gv = jnp.full((L,), cur_g, jnp.int32)
            plsc.addupdate_scatter(acc_c, [gv], jnp.full((L,), rcnt), mask=lane0)

    def process(tau, b):
        x_v = xb[b]
        zero = jnp.zeros((L,), jnp.float32)
        init = (jnp.int32(-1), jnp.float32(0.0)) + (zero,) * (2 * NJ)

        def grp(i16, carry):
            cur_g, rcnt = carry[0], carry[1]
            svecs, qvecs = carry[2:2 + NJ], carry[2 + NJ:]
            iv = idx_v[pl.ds(tau * K + i16 * L, L)]
            g_first = _splat(iv, 0)
            uniform = jnp.all(g_first == _splat(iv, L - 1))
            gf = jnp.max(g_first)
            same = jnp.logical_and(uniform, gf == cur_g)

            ts, tq = [], []
            for j in range(NJ):
                vs = [x_v[i16 * L + r, pl.ds(c0 + j * L, L)] for r in range(L)]
                ts.append(_tree(vs))
                tq.append(_tree([v * v for v in vs]))

            @pl.when(jnp.logical_and(cur_g >= 0, jnp.logical_not(same)))
            def _():
                _flush(cur_g, rcnt, svecs, qvecs)

            @pl.when(jnp.logical_not(uniform))
            def _():
                for k in range(L):
                    g = _splat(iv, k)
                    r = i16 * L + k
                    g2 = lax.shift_right_logical(g, 1)
                    cbase = lax.shift_left(lax.bitwise_and(g, 1), 6)
                    for j in range(NJ):
                        cols = lax.iota(jnp.int32, L) + j * L + cbase
                        v = x_v[r, pl.ds(c0 + j * L, L)]
                        plsc.addupdate_scatter(acc_s, [g2, cols], v)
                        plsc.addupdate_scatter(acc_q, [g2, cols], v * v)

                    @pl.when(cg == 0)
                    def _():
                        plsc.addupdate_scatter(acc_c, [g], ones16, mask=lane0)

            new_g = jnp.where(uniform, gf, jnp.int32(-1))
            new_cnt = jnp.where(same, rcnt + 16.0,
                                jnp.where(uniform, 16.0, 0.0))
            new_s = [jnp.where(same, svecs[j] + ts[j],
                               jnp.where(uniform, ts[j], zero))
                     for j in range(NJ)]
            new_q = [jnp.where(same, qvecs[j] + tq[j],
                               jnp.where(uniform, tq[j], zero))
                     for j in range(NJ)]
            return (new_g, new_cnt) + tuple(new_s) + tuple(new_q)

        def grp_pl(i16, carry):
            return grp(i16, carry)
        fin = plsc.parallel_loop(0, K // L, carry=init)(grp_pl)

        @pl.when(fin[0] >= 0)
        def _():
            _flush(fin[0], fin[1], fin[2:2 + NJ], fin[2 + NJ:])

    def step(tt, _):
        for b in range(2):
            tau = tt * 2 + b

            @pl.when(tau + 1 < cnt)
            def _():
                issue(tau + 1, 1 - b)

            @pl.when(tau < cnt)
            def _():
                wait(b)
                process(tau, b)
        return None
    lax.fori_loop(0, (TPR + 1) // 2, step, None)

    pltpu.sync_copy(acc_s, psum_hbm.at[wid])
    pltpu.sync_copy(acc_q, psq_hbm.at[wid])

    @pl.when(cg == 0)
    def _():
        pltpu.sync_copy(acc_c, pcnt_hbm.at[rg])


def _affine_body(sum_ref, sq_ref, cnt_ref, w_ref, b_ref, scale_ref, shift_ref):
    c = jnp.sum(cnt_ref[...], axis=0)[:, None]
    scales, shifts = [], []
    for cg in range(NCG):
        s = sum_ref[cg]
        q = sq_ref[cg]
        for rg in range(1, NRG):
            s = s + sum_ref[rg * NCG + cg]
            q = q + sq_ref[rg * NCG + cg]
        mean = s / c
        var = jnp.maximum(q / c - mean * mean, 0.0)
        rstd = lax.rsqrt(var + 1e-6)
        sc = rstd * w_ref[0:1, cg * CD:(cg + 1) * CD]
        scales.append(sc)
        shifts.append(b_ref[0:1, cg * CD:(cg + 1) * CD] - mean * sc)
    scale_ref[...] = jnp.concatenate(scales, axis=1)
    shift_ref[...] = jnp.concatenate(shifts, axis=1)


_affine_params = pl.pallas_call(
    _affine_body,
    out_shape=(
        jax.ShapeDtypeStruct((B, D), jnp.float32),
        jax.ShapeDtypeStruct((B, D), jnp.float32),
    ),
)


def _apply_norm_body(x_hbm, idx_hbm, scale_hbm, shift_hbm, out_hbm,
                     idx_v, x0, x1, s0, s1, t0, t1,
                     semi0, semi1, semo0, semo1):
    cid = lax.axis_index("c")
    sid = lax.axis_index("s")
    wid = cid * NS + sid
    base0 = wid * TPW * K
    cnt = jnp.where(wid < NW - 1, TPW, RESW)
    xb, sb, tb = (x0, x1), (s0, s1), (t0, t1)
    semi = (semi0, semi1)
    semo = (semo0, semo1)

    pltpu.async_copy(idx_hbm.at[pl.ds(base0, RESW * K)],
                     idx_v.at[pl.ds(0, RESW * K)], semi1)
    pltpu.make_async_copy(idx_hbm.at[pl.ds(0, RESW * K)],
                          idx_v.at[pl.ds(0, RESW * K)], semi1).wait()

    @pl.when(wid < NW - 1)
    def _():
        rem = (TPW - RESW) * K
        pltpu.async_copy(idx_hbm.at[pl.ds(base0 + RESW * K, rem)],
                         idx_v.at[pl.ds(RESW * K, rem)], semi1)
        pltpu.make_async_copy(idx_hbm.at[pl.ds(0, rem)],
                              idx_v.at[pl.ds(RESW * K, rem)], semi1).wait()

    W = 16

    def tile_window(tau):
        iv0 = idx_v[pl.ds(tau * K, L)]
        ivl = idx_v[pl.ds(tau * K + K - L, L)]
        first = jnp.max(_splat(iv0, 0))
        last = jnp.max(_splat(ivl, L - 1))
        firstc = jnp.minimum(jnp.bitwise_and(first, -8), B - W)
        return firstc, last - firstc < W

    def issue(tau, b):
        rows = pl.ds(base0 + tau * K, K)
        pltpu.async_copy(x_hbm.at[rows], xb[b], semi[b])
        firstc, ok = tile_window(tau)
        fc = pl.multiple_of(firstc, 8)

        @pl.when(ok)
        def _():
            pltpu.async_copy(scale_hbm.at[pl.ds(fc, W)],
                             sb[b].at[pl.ds(0, W)], semi[b])
            pltpu.async_copy(shift_hbm.at[pl.ds(fc, W)],
                             tb[b].at[pl.ds(0, W)], semi[b])

        @pl.when(jnp.logical_not(ok))
        def _():
            isl = idx_v.at[pl.ds(tau * K, K)]
            pltpu.async_copy(scale_hbm.at[isl], sb[b], semi[b])
            pltpu.async_copy(shift_hbm.at[isl], tb[b], semi[b])

    def wait_in(tau, b):
        src = x_hbm.at[pl.ds(0, K)]
        pltpu.make_async_copy(src, xb[b], semi[b]).wait()
        _, ok = tile_window(tau)

        @pl.when(ok)
        def _():
            srw = x_hbm.at[pl.ds(0, W)]
            pltpu.make_async_copy(srw, sb[b].at[pl.ds(0, W)], semi[b]).wait()
            pltpu.make_async_copy(srw, tb[b].at[pl.ds(0, W)], semi[b]).wait()

        @pl.when(jnp.logical_not(ok))
        def _():
            pltpu.make_async_copy(src, sb[b], semi[b]).wait()
            pltpu.make_async_copy(src, tb[b], semi[b]).wait()

    def wait_out(b):
        pltpu.make_async_copy(xb[b], out_hbm.at[pl.ds(0, K)], semo[b]).wait()

    issue(0, 0)

    def step(tt, _):
        for b in range(2):
            tau = tt * 2 + b

            @pl.when(tau + 1 < cnt)
            def _():
                @pl.when(tau >= 1)
                def _():
                    wait_out(1 - b)
                issue(tau + 1, 1 - b)

            @pl.when(tau < cnt)
            def _():
                wait_in(tau, b)
                firstc, ok = tile_window(tau)

                @plsc.parallel_loop(0, K, unroll=4)
                def _(i):
                    iv16 = idx_v[pl.ds(tau * K + jnp.bitwise_and(i, -16), L)]
                    gs = jnp.max(_splat(iv16, jnp.bitwise_and(i, 15)))
                    rsel = jnp.where(ok, gs - firstc, i)
                    for j in range(D // L):
                        sl = pl.ds(j * L, L)
                        xb[b][i, sl] = (xb[b][i, sl] * sb[b][rsel, sl]
                                        + tb[b][rsel, sl])
                pltpu.async_copy(xb[b], out_hbm.at[pl.ds(base0 + tau * K, K)],
                                 semo[b])
        return None
    lax.fori_loop(0, (TPW + 1) // 2, step, None)

    wait_out(0)
    wait_out(1)


@jax.jit
def kernel(tensor, weight, bias, batch_index):
    mesh = plsc.VectorSubcoreMesh(
        core_axis_name="c", subcore_axis_name="s", num_cores=NC, num_subcores=NS
    )
    seg_stats = functools.partial(
        pl.kernel,
        out_type=(
            jax.ShapeDtypeStruct((NW, B // 2, 2 * CD), jnp.float32),
            jax.ShapeDtypeStruct((NW, B // 2, 2 * CD), jnp.float32),
            jax.ShapeDtypeStruct((NRG, B), jnp.float32),
        ),
        mesh=mesh,
        compiler_params=pltpu.CompilerParams(needs_layout_passes=False),
        scratch_types=[
            pltpu.VMEM((TPR * K,), jnp.int32),
            pltpu.VMEM((K, D // 2), jnp.float32),
            pltpu.VMEM((K, D // 2), jnp.float32),
            pltpu.VMEM((B // 2, 2 * CD), jnp.float32),
            pltpu.VMEM((B // 2, 2 * CD), jnp.float32),
            pltpu.VMEM((B,), jnp.float32),
            pltpu.SemaphoreType.DMA,
            pltpu.SemaphoreType.DMA,
        ],
    )(_seg_stats_body)
    apply_norm = functools.partial(
        pl.kernel,
        out_type=jax.ShapeDtypeStruct((N, D), jnp.float32),
        mesh=mesh,
        compiler_params=pltpu.CompilerParams(needs_layout_passes=False),
        scratch_types=[
            pltpu.VMEM((TPW * K,), jnp.int32),
            pltpu.VMEM((K, D), jnp.float32),
            pltpu.VMEM((K, D), jnp.float32),
            pltpu.VMEM((K, D), jnp.float32),
            pltpu.VMEM((K, D), jnp.float32),
            pltpu.VMEM((K, D), jnp.float32),
            pltpu.VMEM((K, D), jnp.float32),
            pltpu.SemaphoreType.DMA,
            pltpu.SemaphoreType.DMA,
            pltpu.SemaphoreType.DMA,
            pltpu.SemaphoreType.DMA,
        ],
    )(_apply_norm_body)

    idx = batch_index.astype(jnp.int32)
    psum_raw, psq_raw, pcnt = seg_stats(tensor, idx)
    psum = psum_raw.reshape(NW, B, CD)
    psq = psq_raw.reshape(NW, B, CD)
    scale, shift = _affine_params(psum, psq, pcnt,
                                  weight.reshape(1, D), bias.reshape(1, D))
    return apply_norm(tensor, idx, scale, shift)
